# baseline (speedup 1.0000x reference)
import jax
import jax.numpy as jnp
from jax.experimental import pallas as pl
from jax.experimental.pallas import tpu as pltpu
from jax.sharding import Mesh, PartitionSpec as P

M = 16384
K = 4096
N = 4096

BM = 256


def _mm_kernel(x_ref, w_ref, o_ref):
    o_ref[...] = jax.lax.dot_general(
        x_ref[...].astype(jnp.bfloat16),
        w_ref[...],
        dimension_numbers=(((1,), (1,)), ((), ())),
        preferred_element_type=jnp.float32,
    )


def _local_mm(x, w_local):
    n_local = w_local.shape[0]
    w16 = w_local.astype(jnp.bfloat16)
    return pl.pallas_call(
        _mm_kernel,
        grid=(M // BM,),
        in_specs=[
            pl.BlockSpec((BM, K), lambda i: (i, 0)),
            pl.BlockSpec((n_local, K), lambda i: (0, 0)),
        ],
        out_specs=pl.BlockSpec((BM, n_local), lambda i: (i, 0)),
        out_shape=jax.ShapeDtypeStruct((M, n_local), jnp.float32),
        compiler_params=pltpu.CompilerParams(
            vmem_limit_bytes=128 * 1024 * 1024,
        ),
    )(x, w16)


def kernel(x, W):
    devs = jax.devices()
    n = min(len(devs), 8)
    while n > 1 and N % n:
        n -= 1
    if n == 1:
        return _local_mm(x, W)
    mesh = Mesh(devs[:n], ("d",))
    f = jax.shard_map(
        _local_mm,
        mesh=mesh,
        in_specs=(P(None, None), P("d", None)),
        out_specs=P(None, "d"),
        check_vma=False,
    )
    return f(x, W)
